# baseline (speedup 1.0000x reference)
import functools

import jax
import jax.numpy as jnp
from jax import lax
from jax.experimental import pallas as pl
from jax.experimental.pallas import tpu as pltpu
from jax.experimental.pallas import tpu_sc as plsc

VOCAB = 1000
DIM = 128
BATCH = 16384

NUM_CORES = 2
NUM_SUBCORES = 16
LANES = 16
NUM_WORKERS = NUM_CORES * NUM_SUBCORES
B_PER_W = BATCH // NUM_WORKERS
CHUNK = 128
N_CHUNKS = B_PER_W // CHUNK

_mesh = plsc.VectorSubcoreMesh(core_axis_name="c", subcore_axis_name="s")


@functools.partial(
    pl.kernel,
    mesh=_mesh,
    out_type=jax.ShapeDtypeStruct((BATCH, DIM), jnp.float32),
    scratch_types=[
        pltpu.VMEM((B_PER_W,), jnp.int32),
        pltpu.VMEM((N_CHUNKS, CHUNK), jnp.int32),
        pltpu.VMEM((B_PER_W, DIM), jnp.float32),
        pltpu.SemaphoreType.DMA,
        pltpu.SemaphoreType.DMA,
        pltpu.SemaphoreType.DMA,
        pltpu.SemaphoreType.DMA,
        pltpu.SemaphoreType.DMA,
    ],
)
def _emb_gather(x_hbm, table_hbm, out_hbm, x_v, idx_v, rows_v,
                g0, g1, g2, g3, so):
    wid = lax.axis_index("s") * NUM_CORES + lax.axis_index("c")
    base = wid * B_PER_W
    gsems = (g0, g1, g2, g3)

    pltpu.sync_copy(x_hbm.at[pl.ds(base, B_PER_W)], x_v)

    per_row = CHUNK // LANES
    gcps = []
    for j in range(N_CHUNKS):
        def _remap(i, _, j=j):
            v = x_v[pl.ds(j * CHUNK + i * LANES, LANES)]
            ok = (v >= 0) & (v < VOCAB)
            idx_v[j, pl.ds(i * LANES, LANES)] = jnp.where(ok, v + 1, 0)
            return 0
        lax.fori_loop(0, per_row, _remap, 0)
        gcps.append(
            pltpu.async_copy(
                table_hbm.at[idx_v.at[j]],
                rows_v.at[pl.ds(j * CHUNK, CHUNK)],
                gsems[j],
            )
        )

    ocps = []
    for j in range(N_CHUNKS):
        gcps[j].wait()
        ocps.append(
            pltpu.async_copy(
                rows_v.at[pl.ds(j * CHUNK, CHUNK)],
                out_hbm.at[pl.ds(base + j * CHUNK, CHUNK)],
                so,
            )
        )
    for cp in ocps:
        cp.wait()


def kernel(x, table):
    xf = x.reshape(BATCH).astype(jnp.int32)
    out = _emb_gather(xf, table)
    return out.reshape(BATCH, 1, DIM)


# speedup vs baseline: 1.2207x; 1.2207x over previous
import functools

import jax
import jax.numpy as jnp
from jax import lax
from jax.experimental import pallas as pl
from jax.experimental.pallas import tpu as pltpu
from jax.experimental.pallas import tpu_sc as plsc

VOCAB = 1000
DIM = 128
BATCH = 16384

NUM_CORES = 2
NUM_SUBCORES = 16
LANES = 16
NUM_WORKERS = NUM_CORES * NUM_SUBCORES
B_PER_W = BATCH // NUM_WORKERS
CHUNK = 128
N_CHUNKS = B_PER_W // CHUNK

ROWS_PAD = 1024
STRIPE = ROWS_PAD // NUM_SUBCORES

_mesh = plsc.VectorSubcoreMesh(core_axis_name="c", subcore_axis_name="s")


@functools.partial(
    pl.kernel,
    mesh=_mesh,
    out_type=jax.ShapeDtypeStruct((BATCH, DIM), jnp.float32),
    scratch_types=[
        pltpu.VMEM_SHARED((ROWS_PAD, DIM), jnp.float32),
        pltpu.VMEM((B_PER_W,), jnp.int32),
        pltpu.VMEM((N_CHUNKS, CHUNK), jnp.int32),
        pltpu.VMEM((B_PER_W, DIM), jnp.float32),
        pltpu.SemaphoreType.DMA,
        pltpu.SemaphoreType.DMA,
        pltpu.SemaphoreType.DMA,
        pltpu.SemaphoreType.DMA,
        pltpu.SemaphoreType.DMA,
    ],
)
def _emb_gather(x_hbm, table_hbm, out_hbm, table_sh, x_v, idx_v, rows_v,
                g0, g1, g2, g3, so):
    sid = lax.axis_index("s")
    wid = sid * NUM_CORES + lax.axis_index("c")
    base = wid * B_PER_W
    gsems = (g0, g1, g2, g3)

    row0 = sid * STRIPE
    pltpu.sync_copy(table_hbm.at[pl.ds(row0, STRIPE)],
                    table_sh.at[pl.ds(row0, STRIPE)])

    pltpu.sync_copy(x_hbm.at[pl.ds(base, B_PER_W)], x_v)

    per_row = CHUNK // LANES
    for j in range(N_CHUNKS):
        def _remap(i, _, j=j):
            v = x_v[pl.ds(j * CHUNK + i * LANES, LANES)]
            ok = (v >= 0) & (v < VOCAB)
            idx_v[j, pl.ds(i * LANES, LANES)] = jnp.where(ok, v + 1, 0)
            return 0
        lax.fori_loop(0, per_row, _remap, 0)

    plsc.subcore_barrier()

    gcps = []
    for j in range(N_CHUNKS):
        gcps.append(
            pltpu.async_copy(
                table_sh.at[idx_v.at[j]],
                rows_v.at[pl.ds(j * CHUNK, CHUNK)],
                gsems[j],
            )
        )

    ocps = []
    for j in range(N_CHUNKS):
        gcps[j].wait()
        ocps.append(
            pltpu.async_copy(
                rows_v.at[pl.ds(j * CHUNK, CHUNK)],
                out_hbm.at[pl.ds(base + j * CHUNK, CHUNK)],
                so,
            )
        )
    for cp in ocps:
        cp.wait()


def kernel(x, table):
    xf = x.reshape(BATCH).astype(jnp.int32)
    tp = jnp.pad(table, ((0, ROWS_PAD - table.shape[0]), (0, 0)))
    out = _emb_gather(xf, tp)
    return out.reshape(BATCH, 1, DIM)


# speedup vs baseline: 1.2413x; 1.0169x over previous
import functools

import jax
import jax.numpy as jnp
from jax import lax
from jax.experimental import pallas as pl
from jax.experimental.pallas import tpu as pltpu
from jax.experimental.pallas import tpu_sc as plsc

VOCAB = 1000
DIM = 128
BATCH = 16384

NUM_CORES = 2
NUM_SUBCORES = 16
LANES = 16
NUM_WORKERS = NUM_CORES * NUM_SUBCORES
B_PER_W = BATCH // NUM_WORKERS
CHUNK = 64
N_CHUNKS = B_PER_W // CHUNK

ROWS = VOCAB + 1
ROWS_PAD = 1024
STRIPE = 64
N_STRIPES = 15
TAIL0 = N_STRIPES * STRIPE
TAIL = ROWS - TAIL0
TAIL_PAD = 48

_mesh = plsc.VectorSubcoreMesh(core_axis_name="c", subcore_axis_name="s")


@functools.partial(
    pl.kernel,
    mesh=_mesh,
    out_type=jax.ShapeDtypeStruct((BATCH, DIM), jnp.float32),
    scratch_types=[
        pltpu.VMEM_SHARED((ROWS_PAD, DIM), jnp.float32),
        pltpu.VMEM((B_PER_W,), jnp.int32),
        pltpu.VMEM((N_CHUNKS, CHUNK), jnp.int32),
        pltpu.VMEM((B_PER_W, DIM), jnp.float32),
        pltpu.VMEM((TAIL_PAD,), jnp.int32),
        pltpu.VMEM((TAIL_PAD, DIM), jnp.float32),
        pltpu.SemaphoreType.DMA,
        pltpu.SemaphoreType.DMA,
        pltpu.SemaphoreType.DMA,
        pltpu.SemaphoreType.DMA,
        pltpu.SemaphoreType.DMA,
    ],
)
def _emb_gather(x_hbm, table_hbm, out_hbm, table_sh, x_v, idx_v, rows_v,
                tidx_v, trows_v, st, sx, ga, gb, so):
    sid = lax.axis_index("s")
    wid = sid * NUM_CORES + lax.axis_index("c")
    base = wid * B_PER_W
    gsems = (ga, gb)

    xcp = pltpu.make_async_copy(x_hbm.at[pl.ds(base, B_PER_W)], x_v, sx)
    xcp.start()

    @pl.when(sid < N_STRIPES)
    def _():
        row0 = sid * STRIPE
        pltpu.async_copy(table_hbm.at[pl.ds(row0, STRIPE)],
                         table_sh.at[pl.ds(row0, STRIPE)], st).wait()

    @pl.when(sid == N_STRIPES)
    def _():
        for j in range(TAIL_PAD // LANES):
            v = jax.lax.iota(jnp.int32, LANES) + (TAIL0 + j * LANES)
            tidx_v[pl.ds(j * LANES, LANES)] = jnp.minimum(v, ROWS - 1)
        pltpu.async_copy(table_hbm.at[tidx_v], trows_v, st).wait()
        pltpu.async_copy(trows_v, table_sh.at[tidx_v], st).wait()

    xcp.wait()
    n_vec = CHUNK // LANES
    for j in range(N_CHUNKS):
        def _remap(i, _, j=j):
            v = x_v[pl.ds(j * CHUNK + i * LANES, LANES)]
            ok = (v >= 0) & (v < VOCAB)
            idx_v[j, pl.ds(i * LANES, LANES)] = jnp.where(ok, v + 1, 0)
            return 0
        lax.fori_loop(0, n_vec, _remap, 0)

    plsc.subcore_barrier()

    ocps = []
    for j in range(N_CHUNKS):
        pltpu.async_copy(
            table_sh.at[idx_v.at[j]],
            rows_v.at[pl.ds(j * CHUNK, CHUNK)],
            gsems[j % 2],
        ).wait()
        ocps.append(
            pltpu.async_copy(
                rows_v.at[pl.ds(j * CHUNK, CHUNK)],
                out_hbm.at[pl.ds(base + j * CHUNK, CHUNK)],
                so,
            )
        )
    for cp in ocps:
        cp.wait()


def kernel(x, table):
    xf = x.reshape(BATCH).astype(jnp.int32)
    out = _emb_gather(xf, table)
    return out.reshape(BATCH, 1, DIM)


# speedup vs baseline: 1.2527x; 1.0091x over previous
import functools

import jax
import jax.numpy as jnp
from jax import lax
from jax.experimental import pallas as pl
from jax.experimental.pallas import tpu as pltpu
from jax.experimental.pallas import tpu_sc as plsc

VOCAB = 1000
DIM = 128
BATCH = 16384

NUM_CORES = 2
NUM_SUBCORES = 16
LANES = 16
NUM_WORKERS = NUM_CORES * NUM_SUBCORES
B_PER_W = BATCH // NUM_WORKERS
CHUNK = 64
N_CHUNKS = B_PER_W // CHUNK

ROWS = VOCAB + 1
ROWS_PAD = 1024
STRIPE = 64
N_STRIPES = 15
TAIL0 = N_STRIPES * STRIPE
TAIL = ROWS - TAIL0
TAIL_PAD = 48

_mesh = plsc.VectorSubcoreMesh(core_axis_name="c", subcore_axis_name="s")


@functools.partial(
    pl.kernel,
    mesh=_mesh,
    out_type=jax.ShapeDtypeStruct((BATCH, DIM), jnp.float32),
    scratch_types=[
        pltpu.VMEM_SHARED((ROWS_PAD, DIM), jnp.float32),
        pltpu.VMEM((B_PER_W,), jnp.int32),
        pltpu.VMEM((N_CHUNKS, CHUNK), jnp.int32),
        pltpu.VMEM((B_PER_W, DIM), jnp.float32),
        pltpu.VMEM((TAIL_PAD,), jnp.int32),
        pltpu.VMEM((TAIL_PAD, DIM), jnp.float32),
        pltpu.SemaphoreType.DMA,
        pltpu.SemaphoreType.DMA,
        pltpu.SemaphoreType.DMA,
        pltpu.SemaphoreType.DMA,
        pltpu.SemaphoreType.DMA,
    ],
)
def _emb_gather(x_hbm, table_hbm, out_hbm, table_sh, x_v, idx_v, rows_v,
                tidx_v, trows_v, st, sx, ga, gb, so):
    sid = lax.axis_index("s")
    wid = sid * NUM_CORES + lax.axis_index("c")
    base = wid * B_PER_W
    gsems = (ga, gb)

    xcp = pltpu.make_async_copy(x_hbm.at[pl.ds(base, B_PER_W)], x_v, sx)
    xcp.start()

    @pl.when(sid < N_STRIPES)
    def _():
        row0 = sid * STRIPE
        pltpu.async_copy(table_hbm.at[pl.ds(row0, STRIPE)],
                         table_sh.at[pl.ds(row0, STRIPE)], st).wait()

    @pl.when(sid == N_STRIPES)
    def _():
        def _tidx(j, _):
            v = jax.lax.iota(jnp.int32, LANES) + (TAIL0 + j * LANES)
            tidx_v[pl.ds(j * LANES, LANES)] = jnp.minimum(v, ROWS - 1)
            return 0
        lax.fori_loop(0, TAIL_PAD // LANES, _tidx, 0)
        pltpu.async_copy(table_hbm.at[tidx_v], trows_v, st).wait()
        pltpu.async_copy(trows_v, table_sh.at[tidx_v], st).wait()

    xcp.wait()

    def _remap(i, _):
        v = x_v[pl.ds(i * LANES, LANES)]
        ok = (v >= 0) & (v < VOCAB)
        idx_v[i // (CHUNK // LANES), pl.ds((i % (CHUNK // LANES)) * LANES, LANES)] = (
            jnp.where(ok, v + 1, 0))
        return 0
    lax.fori_loop(0, B_PER_W // LANES, _remap, 0)

    plsc.subcore_barrier()

    def _chunk(j, _):
        pltpu.async_copy(
            table_sh.at[idx_v.at[j]],
            rows_v.at[pl.ds(j * CHUNK, CHUNK)],
            ga,
        ).wait()
        pltpu.make_async_copy(
            rows_v.at[pl.ds(j * CHUNK, CHUNK)],
            out_hbm.at[pl.ds(base + j * CHUNK, CHUNK)],
            so,
        ).start()
        return 0
    lax.fori_loop(0, N_CHUNKS, _chunk, 0)

    pltpu.make_async_copy(rows_v, out_hbm.at[pl.ds(base, B_PER_W)], so).wait()


def kernel(x, table):
    xf = x.reshape(BATCH).astype(jnp.int32)
    out = _emb_gather(xf, table)
    return out.reshape(BATCH, 1, DIM)


# speedup vs baseline: 1.2546x; 1.0016x over previous
import functools

import jax
import jax.numpy as jnp
from jax import lax
from jax.experimental import pallas as pl
from jax.experimental.pallas import tpu as pltpu
from jax.experimental.pallas import tpu_sc as plsc

VOCAB = 1000
DIM = 128
BATCH = 16384

NUM_CORES = 2
NUM_SUBCORES = 16
LANES = 16
NUM_WORKERS = NUM_CORES * NUM_SUBCORES
B_PER_W = BATCH // NUM_WORKERS
CHUNK = 128
N_CHUNKS = B_PER_W // CHUNK

ROWS = VOCAB + 1
ROWS_PAD = 1024
STRIPE = 64
N_STRIPES = 15
TAIL0 = N_STRIPES * STRIPE
TAIL = ROWS - TAIL0
TAIL_PAD = 48

_mesh = plsc.VectorSubcoreMesh(core_axis_name="c", subcore_axis_name="s")


@functools.partial(
    pl.kernel,
    mesh=_mesh,
    out_type=jax.ShapeDtypeStruct((BATCH, DIM), jnp.float32),
    scratch_types=[
        pltpu.VMEM_SHARED((ROWS_PAD, DIM), jnp.float32),
        pltpu.VMEM((B_PER_W,), jnp.int32),
        pltpu.VMEM((N_CHUNKS, CHUNK), jnp.int32),
        pltpu.VMEM((B_PER_W, DIM), jnp.float32),
        pltpu.VMEM((TAIL_PAD,), jnp.int32),
        pltpu.VMEM((TAIL_PAD, DIM), jnp.float32),
        pltpu.SemaphoreType.DMA,
        pltpu.SemaphoreType.DMA,
        pltpu.SemaphoreType.DMA,
        pltpu.SemaphoreType.DMA,
        pltpu.SemaphoreType.DMA,
    ],
)
def _emb_gather(x_hbm, table_hbm, out_hbm, table_sh, x_v, idx_v, rows_v,
                tidx_v, trows_v, st, sx, ga, gb, so):
    sid = lax.axis_index("s")
    wid = sid * NUM_CORES + lax.axis_index("c")
    base = wid * B_PER_W
    gsems = (ga, gb)

    xcp = pltpu.make_async_copy(x_hbm.at[pl.ds(base, B_PER_W)], x_v, sx)
    xcp.start()

    @pl.when(sid < N_STRIPES)
    def _():
        row0 = sid * STRIPE
        pltpu.async_copy(table_hbm.at[pl.ds(row0, STRIPE)],
                         table_sh.at[pl.ds(row0, STRIPE)], st).wait()

    @pl.when(sid == N_STRIPES)
    def _():
        def _tidx(j, _):
            v = jax.lax.iota(jnp.int32, LANES) + (TAIL0 + j * LANES)
            tidx_v[pl.ds(j * LANES, LANES)] = jnp.minimum(v, ROWS - 1)
            return 0
        lax.fori_loop(0, TAIL_PAD // LANES, _tidx, 0)
        pltpu.async_copy(table_hbm.at[tidx_v], trows_v, st).wait()
        pltpu.async_copy(trows_v, table_sh.at[tidx_v], st).wait()

    xcp.wait()

    def _remap(i, _):
        v = x_v[pl.ds(i * LANES, LANES)]
        ok = (v >= 0) & (v < VOCAB)
        idx_v[i // (CHUNK // LANES), pl.ds((i % (CHUNK // LANES)) * LANES, LANES)] = (
            jnp.where(ok, v + 1, 0))
        return 0
    lax.fori_loop(0, B_PER_W // LANES, _remap, 0)

    plsc.subcore_barrier()

    def _chunk(j, _):
        pltpu.async_copy(
            table_sh.at[idx_v.at[j]],
            rows_v.at[pl.ds(j * CHUNK, CHUNK)],
            ga,
        ).wait()
        pltpu.make_async_copy(
            rows_v.at[pl.ds(j * CHUNK, CHUNK)],
            out_hbm.at[pl.ds(base + j * CHUNK, CHUNK)],
            so,
        ).start()
        return 0
    lax.fori_loop(0, N_CHUNKS, _chunk, 0)

    pltpu.make_async_copy(rows_v, out_hbm.at[pl.ds(base, B_PER_W)], so).wait()


def kernel(x, table):
    xf = x.reshape(BATCH).astype(jnp.int32)
    out = _emb_gather(xf, table)
    return out.reshape(BATCH, 1, DIM)
